# baseline (speedup 1.0000x reference)
import jax
import jax.numpy as jnp
from jax.experimental import pallas as pl
from jax.experimental.pallas import tpu as pltpu


def _fused_net_kernel(xe_ref, xo_ref, B1_ref, B2_ref, b2_ref,
                      f1_ref, f1b_ref, f2_ref, f2b_ref, o_ref):
    bn = xe_ref.shape[1]

    xe = xe_ref[...]
    xo = xo_ref[...]
    ones = jnp.ones((13, bn, 1), jnp.float32)
    xce = jnp.concatenate([xe[0:13], xo[0:13], xe[1:14], ones],
                          axis=-1).reshape(13 * bn, 85).astype(jnp.bfloat16)
    xco = jnp.concatenate([xo[0:13], xe[1:14], xo[1:14], ones],
                          axis=-1).reshape(13 * bn, 85).astype(jnp.bfloat16)
    h1e = jnp.maximum(jnp.dot(xce, B1_ref[...],
                              preferred_element_type=jnp.float32),
                      0.0).astype(jnp.bfloat16)
    h1o = jnp.maximum(jnp.dot(xco, B1_ref[...],
                              preferred_element_type=jnp.float32),
                      0.0).astype(jnp.bfloat16)

    slabs = {}
    for p in range(2):
        for dh in range(3):
            src = h1o if (p + dh) % 2 else h1e
            s = (p + dh) // 2
            slabs[(p, dh)] = src[s * bn:(s + 12) * bn]

    groups = []
    for g in range(6):
        lhs = jnp.concatenate(
            [jnp.concatenate([slabs[(p, dh)][:, 256 * g:256 * g + 256]
                              for dh in range(3)], axis=1)
             for p in range(2)], axis=0)
        a = jnp.dot(lhs, B2_ref[g], preferred_element_type=jnp.float32)
        aw = jnp.maximum(a[:, 0:128], a[:, 128:256])
        groups.append(jnp.maximum(aw[0:12 * bn], aw[12 * bn:]))
    pooled = jnp.concatenate(groups, axis=-1)
    feat = jnp.maximum(pooled + b2_ref[...], 0.0).astype(jnp.bfloat16)

    hid = jnp.dot(feat[0:bn], f1_ref[0],
                  preferred_element_type=jnp.float32)
    for k in range(1, 12):
        hid = hid + jnp.dot(feat[k * bn:(k + 1) * bn], f1_ref[k],
                            preferred_element_type=jnp.float32)
    hid = jnp.maximum(hid + f1b_ref[...], 0.0).astype(jnp.bfloat16)
    logits = jnp.dot(hid, f2_ref[...],
                     preferred_element_type=jnp.float32) + f2b_ref[...]
    m = jnp.max(logits, axis=-1, keepdims=True)
    shifted = logits - m
    lse = jnp.log(jnp.sum(jnp.exp(shifted), axis=-1, keepdims=True))
    o_ref[...] = (shifted - lse).astype(o_ref.dtype)


def kernel(c1_w, c1_b, c2_w, c2_b, f1_w, f1_b, f2_w, f2_b, x):
    N = x.shape[0]
    xr = x.reshape(N, 28, 28)
    xe = xr[:, 0::2, :].transpose(1, 0, 2)
    xo = xr[:, 1::2, :].transpose(1, 0, 2)

    E1 = jnp.stack([jnp.eye(28, 26, k=-t, dtype=jnp.float32)
                    for t in range(3)])
    B1 = jnp.einsum('twv,dtc->dwvc', E1, c1_w.reshape(3, 3, 32))
    B1 = B1.reshape(84, 26, 32)
    pos = jnp.arange(48)
    wi_of_pos = 4 * (pos // 8) + pos % 8
    valid = (wi_of_pos <= 25).astype(jnp.float32)
    B1x = B1[:, jnp.clip(wi_of_pos, 0, 25), :] * valid[None, :, None]
    B1x = B1x.reshape(84, 1536)
    b1row = jnp.tile(c1_b, (1, 48))
    B1x = jnp.concatenate([B1x, b1row], axis=0).astype(jnp.bfloat16)

    pl_i = jnp.arange(8)[:, None, None, None]
    wpl_i = jnp.arange(2)[None, :, None, None]
    wpar_i = jnp.arange(2)[None, None, :, None]
    dw_i = jnp.arange(3)[None, None, None, :]
    D = (pl_i == 2 * wpl_i + wpar_i + dw_i).astype(jnp.float32)
    w2r = c2_w.reshape(3, 3, 32, 64)
    B2c = jnp.einsum('pwqd,hdcu->hpcqwu', D, w2r)
    B2c = B2c.reshape(3, 256, 256)
    B2core = B2c.reshape(768, 256)
    rmask = (jnp.arange(256) < 192).astype(jnp.float32)
    rmask = jnp.tile(rmask, (3,))[:, None]
    B2x = jnp.stack([B2core] * 5 + [B2core * rmask], axis=0)
    B2x = B2x.astype(jnp.bfloat16)

    b2t = jnp.tile(c2_b, (1, 12))
    f1 = f1_w.astype(jnp.bfloat16).reshape(12, 768, 128)
    f2 = f2_w.astype(jnp.bfloat16)

    bn = 128
    grid = (N // bn,)
    return pl.pallas_call(
        _fused_net_kernel,
        out_shape=jax.ShapeDtypeStruct((N, 10), x.dtype),
        grid=grid,
        in_specs=[
            pl.BlockSpec((14, bn, 28), lambda n: (0, n, 0)),
            pl.BlockSpec((14, bn, 28), lambda n: (0, n, 0)),
            pl.BlockSpec((85, 1536), lambda n: (0, 0)),
            pl.BlockSpec((6, 768, 256), lambda n: (0, 0, 0)),
            pl.BlockSpec((1, 768), lambda n: (0, 0)),
            pl.BlockSpec((12, 768, 128), lambda n: (0, 0, 0)),
            pl.BlockSpec((1, 128), lambda n: (0, 0)),
            pl.BlockSpec((128, 10), lambda n: (0, 0)),
            pl.BlockSpec((1, 10), lambda n: (0, 0)),
        ],
        out_specs=pl.BlockSpec((bn, 10), lambda n: (n, 0)),
        compiler_params=pltpu.CompilerParams(
            dimension_semantics=("parallel",),
            vmem_limit_bytes=64 * 1024 * 1024),
    )(xe, xo, B1x, B2x, b2t, f1, f1_b, f2, f2_b)
